# baseline (speedup 1.0000x reference)
import functools

import jax
import jax.numpy as jnp
from jax import lax
from jax.experimental import pallas as pl
from jax.experimental.pallas import tpu as pltpu
from jax.experimental.pallas import tpu_sc as plsc

N = 10000
E = 320000
D = 128

_NC = 2
_NS = 16
_NW = _NC * _NS
_CHUNK = 80
_PER_W = E // _NW
_T = _PER_W // _CHUNK


_NBUF = 4


_XSTRIP = 624


def _gather_mul_body(x_hbm, ec_pos, ec_neg, out_pos, out_neg,
                     idx, ra, rb, wo, si, sga, sgb, swb):
    sid = lax.axis_index("s")
    wid = sid * _NC + lax.axis_index("c")
    w_base = wid * _PER_W


    def run_set(ec, out):
        def istart(t, b):
            base = w_base + t * _CHUNK
            pltpu.async_copy(ec.at[pl.ds(base, _CHUNK)], idx.at[b, 0],
                             si.at[b])
            pltpu.async_copy(ec.at[pl.ds(E + base, _CHUNK)], idx.at[b, 1],
                             si.at[b])

        def iwait(t, b):
            base = w_base + t * _CHUNK
            pltpu.make_async_copy(ec.at[pl.ds(base, _CHUNK)], idx.at[b, 0],
                                  si.at[b]).wait()
            pltpu.make_async_copy(ec.at[pl.ds(E + base, _CHUNK)],
                                  idx.at[b, 1], si.at[b]).wait()

        def gstart(b):
            pltpu.async_copy(x_hbm.at[idx.at[b, 0]], ra.at[b], sga.at[b])
            pltpu.async_copy(x_hbm.at[idx.at[b, 1]], rb.at[b], sgb.at[b])

        def gwait(b):
            pltpu.make_async_copy(x_hbm.at[idx.at[b, 0]], ra.at[b],
                                  sga.at[b]).wait()
            pltpu.make_async_copy(x_hbm.at[idx.at[b, 1]], rb.at[b],
                                  sgb.at[b]).wait()

        def wbwait(t, b):
            pltpu.make_async_copy(
                wo.at[b], out.at[pl.ds(w_base + t * _CHUNK, _CHUNK)],
                swb.at[b]).wait()

        def body(t, carry):
            b0 = lax.rem(t, _NBUF)
            b2 = lax.rem(t + 2, _NBUF)
            b3 = lax.rem(t + 3, _NBUF)

            @pl.when(t + 3 < _T)
            def _():
                istart(t + 3, b3)

            @pl.when(t + 2 < _T)
            def _():
                iwait(t + 2, b2)

                @pl.when(t >= 2)
                def _():
                    wbwait(t - 2, b2)

                gstart(b2)

            gwait(b0)

            topm = jnp.int32(-65536)
            half = jnp.int32(32768)

            def row_body(r, c):
                for k in range(D // 32):
                    s0 = pl.ds(k * 32, 16)
                    s1 = pl.ds(k * 32 + 16, 16)
                    p0 = ra[b0, r, s0] * rb[b0, r, s0]
                    p1 = ra[b0, r, s1] * rb[b0, r, s1]
                    i0 = lax.bitcast_convert_type(p0, jnp.int32)
                    i1 = lax.bitcast_convert_type(p1, jnp.int32)
                    wo[b0, r, pl.ds(k * 16, 16)] = (
                        ((i1 + half) & topm)
                        | lax.shift_right_logical(i0 + half, 16))
                return c

            lax.fori_loop(0, _CHUNK, row_body, 0, unroll=4)

            pltpu.async_copy(wo.at[b0],
                             out.at[pl.ds(w_base + t * _CHUNK, _CHUNK)],
                             swb.at[b0])
            return carry

        for t in range(3):
            istart(t, t)
        for t in range(2):
            iwait(t, t)
            gstart(t)
        lax.fori_loop(0, _T, body, 0)
        for t in range(_T - _NBUF, _T):
            wbwait(t, t % _NBUF)

    run_set(ec_pos, out_pos)
    run_set(ec_neg, out_neg)


def _gather_mul(x, ei_pos, ei_neg):
    mesh = plsc.VectorSubcoreMesh(core_axis_name="c", subcore_axis_name="s")
    f = functools.partial(
        pl.kernel,
        mesh=mesh,
        out_type=[
            jax.ShapeDtypeStruct((E, D // 2), jnp.int32),
            jax.ShapeDtypeStruct((E, D // 2), jnp.int32),
        ],
        scratch_types=[
            pltpu.VMEM((_NBUF, 2, _CHUNK), jnp.int32),
            pltpu.VMEM((_NBUF, _CHUNK, D), jnp.float32),
            pltpu.VMEM((_NBUF, _CHUNK, D), jnp.float32),
            pltpu.VMEM((_NBUF, _CHUNK, D // 2), jnp.int32),
            pltpu.SemaphoreType.DMA((_NBUF,)),
            pltpu.SemaphoreType.DMA((_NBUF,)),
            pltpu.SemaphoreType.DMA((_NBUF,)),
            pltpu.SemaphoreType.DMA((_NBUF,)),
        ],
    )(_gather_mul_body)
    return f(x, ei_pos.reshape(2 * E), ei_neg.reshape(2 * E))


_B = 3200
_G = E // _B


def _decode_body(ep_ref, en_ref, wcat_ref, b1_ref, be1_ref, w2t_ref, b2_ref,
                 we2t_ref, be2_ref, attr_ref, pos_ref, neg_ref):
    wcat = wcat_ref[...]
    we2t = we2t_ref[...]
    be2 = be2_ref[...]
    h = jnp.maximum(ep_ref[...], jnp.bfloat16(0))
    a = lax.dot_general(h, wcat, (((1,), (0,)), ((), ())),
                        preferred_element_type=jnp.float32)
    a1 = jnp.maximum(a[:, :D] + b1_ref[...], 0.0)
    attr_ref[...] = jax.nn.sigmoid(jnp.dot(a1, w2t_ref[...]) + b2_ref[...])
    ae = jnp.maximum(a[:, D:] + be1_ref[...], 0.0)
    pos_ref[...] = jax.nn.sigmoid(jnp.dot(ae, we2t) + be2)
    hn = jnp.maximum(en_ref[...], jnp.bfloat16(0))
    an = jnp.maximum(
        lax.dot_general(hn, wcat[:, D:], (((1,), (0,)), ((), ())),
                        preferred_element_type=jnp.float32) + be1_ref[...],
        0.0)
    neg_ref[...] = jax.nn.sigmoid(jnp.dot(an, we2t) + be2)


def _decode(em_pos, em_neg, wcat_t, b1r, be1r, w2t8, b2r, we2t8, be2r):
    return pl.pallas_call(
        _decode_body,
        grid=(_G,),
        in_specs=[
            pl.BlockSpec((_B, D), lambda i: (i, 0)),
            pl.BlockSpec((_B, D), lambda i: (i, 0)),
            pl.BlockSpec((D, 2 * D), lambda i: (0, 0)),
            pl.BlockSpec((1, D), lambda i: (0, 0)),
            pl.BlockSpec((1, D), lambda i: (0, 0)),
            pl.BlockSpec((D, 8), lambda i: (0, 0)),
            pl.BlockSpec((1, 8), lambda i: (0, 0)),
            pl.BlockSpec((D, 8), lambda i: (0, 0)),
            pl.BlockSpec((1, 1), lambda i: (0, 0)),
        ],
        out_specs=[
            pl.BlockSpec((_B, 8), lambda i: (i, 0)),
            pl.BlockSpec((_B, 8), lambda i: (i, 0)),
            pl.BlockSpec((_B, 8), lambda i: (i, 0)),
        ],
        out_shape=[
            jax.ShapeDtypeStruct((E, 8), jnp.float32),
            jax.ShapeDtypeStruct((E, 8), jnp.float32),
            jax.ShapeDtypeStruct((E, 8), jnp.float32),
        ],
        compiler_params=pltpu.CompilerParams(
            dimension_semantics=("arbitrary",),
        ),
    )(em_pos, em_neg, wcat_t, b1r, be1r, w2t8, b2r, we2t8, be2r)


_PERM = [32 * k + i + 16 * h
         for k in range(D // 32) for i in range(16) for h in range(2)]


def kernel(x, edge_index, edge_index_neg, W1, b1, W2, b2, We1, be1, We2, be2):
    emp_i, emn_i = _gather_mul(x, edge_index, edge_index_neg)
    em_pos = lax.bitcast_convert_type(emp_i, jnp.bfloat16).reshape(E, D)
    em_neg = lax.bitcast_convert_type(emn_i, jnp.bfloat16).reshape(E, D)

    wcat_t = jnp.concatenate([W1, We1], axis=0).T.astype(jnp.bfloat16)
    wcat_t = wcat_t[jnp.array(_PERM), :]
    w2t8 = jnp.pad(W2, ((0, 1), (0, 0))).T
    b2r = jnp.pad(b2, (0, 1)).reshape(1, 8)
    we2t8 = jnp.pad(We2, ((0, 7), (0, 0))).T
    attr8, pos8, neg8 = _decode(
        em_pos, em_neg, wcat_t, b1.reshape(1, D), be1.reshape(1, D),
        w2t8, b2r, we2t8, be2.reshape(1, 1))
    return attr8[:, :7], pos8[:, 0], neg8[:, 0]


# speedup vs baseline: 1.1416x; 1.1416x over previous
import functools

import jax
import jax.numpy as jnp
from jax import lax
from jax.experimental import pallas as pl
from jax.experimental.pallas import tpu as pltpu
from jax.experimental.pallas import tpu_sc as plsc

N = 10000
E = 320000
D = 128

_NC = 2
_NS = 16
_NW = _NC * _NS
_CHUNK = 80
_PER_W = E // _NW
_T = _PER_W // _CHUNK


_NBUF = 4


_XSTRIP = 624


def _gather_mul_body(x_hbm, ec_pos, ec_neg, out_pos, out_neg,
                     idx, ra, rb, wo, si, sga, sgb, swb):
    sid = lax.axis_index("s")
    wid = sid * _NC + lax.axis_index("c")
    w_base = wid * _PER_W


    def run_set(ec, out):
        def istart(t, b):
            base = w_base + t * _CHUNK
            pltpu.async_copy(ec.at[pl.ds(base, _CHUNK)], idx.at[b, 0],
                             si.at[b])
            pltpu.async_copy(ec.at[pl.ds(E + base, _CHUNK)], idx.at[b, 1],
                             si.at[b])

        def iwait(t, b):
            base = w_base + t * _CHUNK
            pltpu.make_async_copy(ec.at[pl.ds(base, _CHUNK)], idx.at[b, 0],
                                  si.at[b]).wait()
            pltpu.make_async_copy(ec.at[pl.ds(E + base, _CHUNK)],
                                  idx.at[b, 1], si.at[b]).wait()

        def gstart(b):
            pltpu.async_copy(x_hbm.at[idx.at[b, 0]], ra.at[b], sga.at[b])
            pltpu.async_copy(x_hbm.at[idx.at[b, 1]], rb.at[b], sgb.at[b])

        def gwait(b):
            pltpu.make_async_copy(x_hbm.at[idx.at[b, 0]], ra.at[b],
                                  sga.at[b]).wait()
            pltpu.make_async_copy(x_hbm.at[idx.at[b, 1]], rb.at[b],
                                  sgb.at[b]).wait()

        def wbwait(t, b):
            pltpu.make_async_copy(
                wo.at[b], out.at[pl.ds(w_base + t * _CHUNK, _CHUNK)],
                swb.at[b]).wait()

        def body(t, carry):
            b0 = lax.rem(t, _NBUF)
            b2 = lax.rem(t + 2, _NBUF)
            b3 = lax.rem(t + 3, _NBUF)

            @pl.when(t + 3 < _T)
            def _():
                istart(t + 3, b3)

            @pl.when(t + 2 < _T)
            def _():
                iwait(t + 2, b2)

                @pl.when(t >= 2)
                def _():
                    wbwait(t - 2, b2)

                gstart(b2)

            gwait(b0)

            topm = jnp.int32(-65536)
            half = jnp.int32(32768)

            @plsc.parallel_loop(0, _CHUNK, unroll=4)
            def row_body(r):
                for k in range(D // 32):
                    s0 = pl.ds(k * 32, 16)
                    s1 = pl.ds(k * 32 + 16, 16)
                    p0 = ra[b0, r, s0] * rb[b0, r, s0]
                    p1 = ra[b0, r, s1] * rb[b0, r, s1]
                    i0 = lax.bitcast_convert_type(p0, jnp.int32)
                    i1 = lax.bitcast_convert_type(p1, jnp.int32)
                    wo[b0, r, pl.ds(k * 16, 16)] = (
                        ((i1 + half) & topm)
                        | lax.shift_right_logical(i0 + half, 16))

            pltpu.async_copy(wo.at[b0],
                             out.at[pl.ds(w_base + t * _CHUNK, _CHUNK)],
                             swb.at[b0])
            return carry

        for t in range(3):
            istart(t, t)
        for t in range(2):
            iwait(t, t)
            gstart(t)
        lax.fori_loop(0, _T, body, 0)
        for t in range(_T - _NBUF, _T):
            wbwait(t, t % _NBUF)

    run_set(ec_pos, out_pos)
    run_set(ec_neg, out_neg)


def _gather_mul(x, ei_pos, ei_neg):
    mesh = plsc.VectorSubcoreMesh(core_axis_name="c", subcore_axis_name="s")
    f = functools.partial(
        pl.kernel,
        mesh=mesh,
        out_type=[
            jax.ShapeDtypeStruct((E, D // 2), jnp.int32),
            jax.ShapeDtypeStruct((E, D // 2), jnp.int32),
        ],
        scratch_types=[
            pltpu.VMEM((_NBUF, 2, _CHUNK), jnp.int32),
            pltpu.VMEM((_NBUF, _CHUNK, D), jnp.float32),
            pltpu.VMEM((_NBUF, _CHUNK, D), jnp.float32),
            pltpu.VMEM((_NBUF, _CHUNK, D // 2), jnp.int32),
            pltpu.SemaphoreType.DMA((_NBUF,)),
            pltpu.SemaphoreType.DMA((_NBUF,)),
            pltpu.SemaphoreType.DMA((_NBUF,)),
            pltpu.SemaphoreType.DMA((_NBUF,)),
        ],
    )(_gather_mul_body)
    return f(x, ei_pos.reshape(2 * E), ei_neg.reshape(2 * E))


_B = 3200
_G = E // _B


def _decode_body(ep_ref, en_ref, wcat_ref, b1_ref, be1_ref, w2t_ref, b2_ref,
                 we2t_ref, be2_ref, attr_ref, pos_ref, neg_ref):
    wcat = wcat_ref[...]
    we2t = we2t_ref[...]
    be2 = be2_ref[...]
    h = jnp.maximum(ep_ref[...], jnp.bfloat16(0))
    a = lax.dot_general(h, wcat, (((1,), (0,)), ((), ())),
                        preferred_element_type=jnp.float32)
    a1 = jnp.maximum(a[:, :D] + b1_ref[...], 0.0)
    attr_ref[...] = jax.nn.sigmoid(jnp.dot(a1, w2t_ref[...]) + b2_ref[...])
    ae = jnp.maximum(a[:, D:] + be1_ref[...], 0.0)
    pos_ref[...] = jax.nn.sigmoid(jnp.dot(ae, we2t) + be2)
    hn = jnp.maximum(en_ref[...], jnp.bfloat16(0))
    an = jnp.maximum(
        lax.dot_general(hn, wcat[:, D:], (((1,), (0,)), ((), ())),
                        preferred_element_type=jnp.float32) + be1_ref[...],
        0.0)
    neg_ref[...] = jax.nn.sigmoid(jnp.dot(an, we2t) + be2)


def _decode(em_pos, em_neg, wcat_t, b1r, be1r, w2t8, b2r, we2t8, be2r):
    return pl.pallas_call(
        _decode_body,
        grid=(_G,),
        in_specs=[
            pl.BlockSpec((_B, D), lambda i: (i, 0)),
            pl.BlockSpec((_B, D), lambda i: (i, 0)),
            pl.BlockSpec((D, 2 * D), lambda i: (0, 0)),
            pl.BlockSpec((1, D), lambda i: (0, 0)),
            pl.BlockSpec((1, D), lambda i: (0, 0)),
            pl.BlockSpec((D, 8), lambda i: (0, 0)),
            pl.BlockSpec((1, 8), lambda i: (0, 0)),
            pl.BlockSpec((D, 8), lambda i: (0, 0)),
            pl.BlockSpec((1, 1), lambda i: (0, 0)),
        ],
        out_specs=[
            pl.BlockSpec((_B, 8), lambda i: (i, 0)),
            pl.BlockSpec((_B, 8), lambda i: (i, 0)),
            pl.BlockSpec((_B, 8), lambda i: (i, 0)),
        ],
        out_shape=[
            jax.ShapeDtypeStruct((E, 8), jnp.float32),
            jax.ShapeDtypeStruct((E, 8), jnp.float32),
            jax.ShapeDtypeStruct((E, 8), jnp.float32),
        ],
        compiler_params=pltpu.CompilerParams(
            dimension_semantics=("arbitrary",),
        ),
    )(em_pos, em_neg, wcat_t, b1r, be1r, w2t8, b2r, we2t8, be2r)


_PERM = [32 * k + i + 16 * h
         for k in range(D // 32) for i in range(16) for h in range(2)]


def kernel(x, edge_index, edge_index_neg, W1, b1, W2, b2, We1, be1, We2, be2):
    emp_i, emn_i = _gather_mul(x, edge_index, edge_index_neg)
    em_pos = lax.bitcast_convert_type(emp_i, jnp.bfloat16).reshape(E, D)
    em_neg = lax.bitcast_convert_type(emn_i, jnp.bfloat16).reshape(E, D)

    wcat_t = jnp.concatenate([W1, We1], axis=0).T.astype(jnp.bfloat16)
    wcat_t = wcat_t[jnp.array(_PERM), :]
    w2t8 = jnp.pad(W2, ((0, 1), (0, 0))).T
    b2r = jnp.pad(b2, (0, 1)).reshape(1, 8)
    we2t8 = jnp.pad(We2, ((0, 7), (0, 0))).T
    attr8, pos8, neg8 = _decode(
        em_pos, em_neg, wcat_t, b1.reshape(1, D), be1.reshape(1, D),
        w2t8, b2r, we2t8, be2.reshape(1, 1))
    return attr8[:, :7], pos8[:, 0], neg8[:, 0]


# speedup vs baseline: 2.5038x; 2.1933x over previous
import functools

import jax
import jax.numpy as jnp
from jax import lax
from jax.experimental import pallas as pl
from jax.experimental.pallas import tpu as pltpu
from jax.experimental.pallas import tpu_sc as plsc

N = 10000
E = 320000
D = 128

_NC = 2
_NS = 16
_NW = _NC * _NS
_CHUNK = 80
_PER_W = E // _NW
_T = _PER_W // _CHUNK
_NBUF = 4


def _gather_body(x_hbm, ec_pos, ec_neg, oa_pos, ob_pos, oa_neg, ob_neg,
                 idx, ra, rb, si, sga, sgb, swa, swb):
    sid = lax.axis_index("s")
    wid = sid * _NC + lax.axis_index("c")
    w_base = wid * _PER_W

    def run_set(ec, oa, ob):
        def istart(t, b):
            base = w_base + t * _CHUNK
            pltpu.async_copy(ec.at[pl.ds(base, _CHUNK)], idx.at[b, 0],
                             si.at[b])
            pltpu.async_copy(ec.at[pl.ds(E + base, _CHUNK)], idx.at[b, 1],
                             si.at[b])

        def iwait(t, b):
            base = w_base + t * _CHUNK
            pltpu.make_async_copy(ec.at[pl.ds(base, _CHUNK)], idx.at[b, 0],
                                  si.at[b]).wait()
            pltpu.make_async_copy(ec.at[pl.ds(E + base, _CHUNK)],
                                  idx.at[b, 1], si.at[b]).wait()

        def gstart(b):
            pltpu.async_copy(x_hbm.at[idx.at[b, 0]], ra.at[b], sga.at[b])
            pltpu.async_copy(x_hbm.at[idx.at[b, 1]], rb.at[b], sgb.at[b])

        def gwait(b):
            pltpu.make_async_copy(x_hbm.at[idx.at[b, 0]], ra.at[b],
                                  sga.at[b]).wait()
            pltpu.make_async_copy(x_hbm.at[idx.at[b, 1]], rb.at[b],
                                  sgb.at[b]).wait()

        def wbwait(t, b):
            sl = pl.ds(w_base + t * _CHUNK, _CHUNK)
            pltpu.make_async_copy(ra.at[b], oa.at[sl], swa.at[b]).wait()
            pltpu.make_async_copy(rb.at[b], ob.at[sl], swb.at[b]).wait()

        def body(t, carry):
            b0 = lax.rem(t, _NBUF)
            b2 = lax.rem(t + 2, _NBUF)
            b3 = lax.rem(t + 3, _NBUF)

            @pl.when(t + 3 < _T)
            def _():
                istart(t + 3, b3)

            @pl.when(t + 2 < _T)
            def _():
                iwait(t + 2, b2)

                @pl.when(t >= 2)
                def _():
                    wbwait(t - 2, b2)

                gstart(b2)

            gwait(b0)
            sl = pl.ds(w_base + t * _CHUNK, _CHUNK)
            pltpu.async_copy(ra.at[b0], oa.at[sl], swa.at[b0])
            pltpu.async_copy(rb.at[b0], ob.at[sl], swb.at[b0])
            return carry

        for t in range(3):
            istart(t, t)
        for t in range(2):
            iwait(t, t)
            gstart(t)
        lax.fori_loop(0, _T, body, 0)
        for t in range(_T - _NBUF, _T):
            wbwait(t, t % _NBUF)

    run_set(ec_pos, oa_pos, ob_pos)
    run_set(ec_neg, oa_neg, ob_neg)


def _gather(x, ei_pos, ei_neg):
    mesh = plsc.VectorSubcoreMesh(core_axis_name="c", subcore_axis_name="s")
    f = functools.partial(
        pl.kernel,
        mesh=mesh,
        out_type=[jax.ShapeDtypeStruct((E, D), jnp.float32)] * 4,
        scratch_types=[
            pltpu.VMEM((_NBUF, 2, _CHUNK), jnp.int32),
            pltpu.VMEM((_NBUF, _CHUNK, D), jnp.float32),
            pltpu.VMEM((_NBUF, _CHUNK, D), jnp.float32),
            pltpu.SemaphoreType.DMA((_NBUF,)),
            pltpu.SemaphoreType.DMA((_NBUF,)),
            pltpu.SemaphoreType.DMA((_NBUF,)),
            pltpu.SemaphoreType.DMA((_NBUF,)),
            pltpu.SemaphoreType.DMA((_NBUF,)),
        ],
    )(_gather_body)
    return f(x, ei_pos.reshape(2 * E), ei_neg.reshape(2 * E))


_B = 3200
_G = E // _B


def _decode_body(xap_ref, xbp_ref, xan_ref, xbn_ref, wcat_ref, b1_ref,
                 be1_ref, w2t_ref, b2_ref, we2t_ref, be2_ref,
                 attr_ref, pos_ref, neg_ref):
    wcat = wcat_ref[...]
    we2t = we2t_ref[...]
    be2 = be2_ref[...]
    h = jnp.maximum(xap_ref[...] * xbp_ref[...], 0.0)
    a = jnp.dot(h, wcat)
    a1 = jnp.maximum(a[:, :D] + b1_ref[...], 0.0)
    attr_ref[...] = jax.nn.sigmoid(jnp.dot(a1, w2t_ref[...]) + b2_ref[...])
    ae = jnp.maximum(a[:, D:] + be1_ref[...], 0.0)
    pos_ref[...] = jax.nn.sigmoid(jnp.dot(ae, we2t) + be2)
    hn = jnp.maximum(xan_ref[...] * xbn_ref[...], 0.0)
    an = jnp.maximum(jnp.dot(hn, wcat[:, D:]) + be1_ref[...], 0.0)
    neg_ref[...] = jax.nn.sigmoid(jnp.dot(an, we2t) + be2)


def _decode(xap, xbp, xan, xbn, wcat_t, b1r, be1r, w2t8, b2r, we2t8, be2r):
    ewise = pl.BlockSpec((_B, D), lambda i: (i, 0))
    return pl.pallas_call(
        _decode_body,
        grid=(_G,),
        in_specs=[
            ewise, ewise, ewise, ewise,
            pl.BlockSpec((D, 2 * D), lambda i: (0, 0)),
            pl.BlockSpec((1, D), lambda i: (0, 0)),
            pl.BlockSpec((1, D), lambda i: (0, 0)),
            pl.BlockSpec((D, 8), lambda i: (0, 0)),
            pl.BlockSpec((1, 8), lambda i: (0, 0)),
            pl.BlockSpec((D, 8), lambda i: (0, 0)),
            pl.BlockSpec((1, 1), lambda i: (0, 0)),
        ],
        out_specs=[
            pl.BlockSpec((_B, 8), lambda i: (i, 0)),
            pl.BlockSpec((_B, 8), lambda i: (i, 0)),
            pl.BlockSpec((_B, 8), lambda i: (i, 0)),
        ],
        out_shape=[
            jax.ShapeDtypeStruct((E, 8), jnp.float32),
            jax.ShapeDtypeStruct((E, 8), jnp.float32),
            jax.ShapeDtypeStruct((E, 8), jnp.float32),
        ],
        compiler_params=pltpu.CompilerParams(
            dimension_semantics=("arbitrary",),
        ),
    )(xap, xbp, xan, xbn, wcat_t, b1r, be1r, w2t8, b2r, we2t8, be2r)


def kernel(x, edge_index, edge_index_neg, W1, b1, W2, b2, We1, be1, We2, be2):
    xap, xbp, xan, xbn = _gather(x, edge_index, edge_index_neg)

    wcat_t = jnp.concatenate([W1, We1], axis=0).T
    w2t8 = jnp.pad(W2, ((0, 1), (0, 0))).T
    b2r = jnp.pad(b2, (0, 1)).reshape(1, 8)
    we2t8 = jnp.pad(We2, ((0, 7), (0, 0))).T
    attr8, pos8, neg8 = _decode(
        xap, xbp, xan, xbn, wcat_t, b1.reshape(1, D), be1.reshape(1, D),
        w2t8, b2r, we2t8, be2.reshape(1, 1))
    return attr8[:, :7], pos8[:, 0], neg8[:, 0]


# speedup vs baseline: 2.5458x; 1.0168x over previous
import functools

import jax
import jax.numpy as jnp
from jax import lax
from jax.experimental import pallas as pl
from jax.experimental.pallas import tpu as pltpu
from jax.experimental.pallas import tpu_sc as plsc

N = 10000
E = 320000
D = 128

_NC = 2
_NS = 16
_NW = _NC * _NS
_CHUNK = 80
_PER_W = E // _NW
_T = _PER_W // _CHUNK
_NBUF = 4


def _gather_body(x_hbm, ec, oa, ob, idx, ra, rb, si, sga, sgb, swa, swb):
    sid = lax.axis_index("s")
    wid = sid * _NC + lax.axis_index("c")
    w_base = wid * _PER_W

    def istart(t, b):
        base = w_base + t * _CHUNK
        pltpu.async_copy(ec.at[pl.ds(base, _CHUNK)], idx.at[b, 0], si.at[b])
        pltpu.async_copy(ec.at[pl.ds(E + base, _CHUNK)], idx.at[b, 1],
                         si.at[b])

    def iwait(t, b):
        base = w_base + t * _CHUNK
        pltpu.make_async_copy(ec.at[pl.ds(base, _CHUNK)], idx.at[b, 0],
                              si.at[b]).wait()
        pltpu.make_async_copy(ec.at[pl.ds(E + base, _CHUNK)], idx.at[b, 1],
                              si.at[b]).wait()

    def gstart(b):
        pltpu.async_copy(x_hbm.at[idx.at[b, 0]], ra.at[b], sga.at[b])
        pltpu.async_copy(x_hbm.at[idx.at[b, 1]], rb.at[b], sgb.at[b])

    def gwait(b):
        pltpu.make_async_copy(x_hbm.at[idx.at[b, 0]], ra.at[b],
                              sga.at[b]).wait()
        pltpu.make_async_copy(x_hbm.at[idx.at[b, 1]], rb.at[b],
                              sgb.at[b]).wait()

    def wbwait(t, b):
        sl = pl.ds(w_base + t * _CHUNK, _CHUNK)
        pltpu.make_async_copy(ra.at[b], oa.at[sl], swa.at[b]).wait()
        pltpu.make_async_copy(rb.at[b], ob.at[sl], swb.at[b]).wait()

    def body(t, carry):
        b0 = lax.rem(t, _NBUF)
        b2 = lax.rem(t + 2, _NBUF)
        b3 = lax.rem(t + 3, _NBUF)

        @pl.when(t + 3 < _T)
        def _():
            istart(t + 3, b3)

        @pl.when(t + 2 < _T)
        def _():
            iwait(t + 2, b2)

            @pl.when(t >= 2)
            def _():
                wbwait(t - 2, b2)

            gstart(b2)

        gwait(b0)
        sl = pl.ds(w_base + t * _CHUNK, _CHUNK)
        pltpu.async_copy(ra.at[b0], oa.at[sl], swa.at[b0])
        pltpu.async_copy(rb.at[b0], ob.at[sl], swb.at[b0])
        return carry

    for t in range(3):
        istart(t, t)
    for t in range(2):
        iwait(t, t)
        gstart(t)
    lax.fori_loop(0, _T, body, 0)
    for t in range(_T - _NBUF, _T):
        wbwait(t, t % _NBUF)


def _gather(x, ei):
    mesh = plsc.VectorSubcoreMesh(core_axis_name="c", subcore_axis_name="s")
    f = functools.partial(
        pl.kernel,
        mesh=mesh,
        out_type=[jax.ShapeDtypeStruct((E, D), jnp.float32)] * 2,
        scratch_types=[
            pltpu.VMEM((_NBUF, 2, _CHUNK), jnp.int32),
            pltpu.VMEM((_NBUF, _CHUNK, D), jnp.float32),
            pltpu.VMEM((_NBUF, _CHUNK, D), jnp.float32),
            pltpu.SemaphoreType.DMA((_NBUF,)),
            pltpu.SemaphoreType.DMA((_NBUF,)),
            pltpu.SemaphoreType.DMA((_NBUF,)),
            pltpu.SemaphoreType.DMA((_NBUF,)),
            pltpu.SemaphoreType.DMA((_NBUF,)),
        ],
    )(_gather_body)
    return f(x, ei.reshape(2 * E))


_B = 4000
_G = E // _B

_EWISE = pl.BlockSpec((_B, D), lambda i: (i, 0))
_HEAD = pl.BlockSpec((_B, 8), lambda i: (i, 0))
_HEAD_SHAPE = jax.ShapeDtypeStruct((E, 8), jnp.float32)


def _decode_pos_body(xa_ref, xb_ref, wcat_ref, b1_ref, be1_ref, w2t_ref,
                     b2_ref, we2t_ref, be2_ref, attr_ref, pos_ref):
    h = jnp.maximum(xa_ref[...] * xb_ref[...], 0.0)
    a = jnp.dot(h, wcat_ref[...])
    a1 = jnp.maximum(a[:, :D] + b1_ref[...], 0.0)
    attr_ref[...] = jax.nn.sigmoid(jnp.dot(a1, w2t_ref[...]) + b2_ref[...])
    ae = jnp.maximum(a[:, D:] + be1_ref[...], 0.0)
    pos_ref[...] = jax.nn.sigmoid(jnp.dot(ae, we2t_ref[...]) + be2_ref[...])


def _decode_neg_body(xa_ref, xb_ref, we1t_ref, be1_ref, we2t_ref, be2_ref,
                     neg_ref):
    h = jnp.maximum(xa_ref[...] * xb_ref[...], 0.0)
    an = jnp.maximum(jnp.dot(h, we1t_ref[...]) + be1_ref[...], 0.0)
    neg_ref[...] = jax.nn.sigmoid(jnp.dot(an, we2t_ref[...]) + be2_ref[...])


def _decode_pos(xa, xb, wcat_t, b1r, be1r, w2t8, b2r, we2t8, be2r):
    return pl.pallas_call(
        _decode_pos_body,
        grid=(_G,),
        in_specs=[
            _EWISE, _EWISE,
            pl.BlockSpec((D, 2 * D), lambda i: (0, 0)),
            pl.BlockSpec((1, D), lambda i: (0, 0)),
            pl.BlockSpec((1, D), lambda i: (0, 0)),
            pl.BlockSpec((D, 8), lambda i: (0, 0)),
            pl.BlockSpec((1, 8), lambda i: (0, 0)),
            pl.BlockSpec((D, 8), lambda i: (0, 0)),
            pl.BlockSpec((1, 1), lambda i: (0, 0)),
        ],
        out_specs=[_HEAD, _HEAD],
        out_shape=[_HEAD_SHAPE, _HEAD_SHAPE],
        compiler_params=pltpu.CompilerParams(
            dimension_semantics=("arbitrary",),
        ),
    )(xa, xb, wcat_t, b1r, be1r, w2t8, b2r, we2t8, be2r)


def _decode_neg(xa, xb, we1t, be1r, we2t8, be2r):
    return pl.pallas_call(
        _decode_neg_body,
        grid=(_G,),
        in_specs=[
            _EWISE, _EWISE,
            pl.BlockSpec((D, D), lambda i: (0, 0)),
            pl.BlockSpec((1, D), lambda i: (0, 0)),
            pl.BlockSpec((D, 8), lambda i: (0, 0)),
            pl.BlockSpec((1, 1), lambda i: (0, 0)),
        ],
        out_specs=[_HEAD],
        out_shape=[_HEAD_SHAPE],
        compiler_params=pltpu.CompilerParams(
            dimension_semantics=("arbitrary",),
        ),
    )(xa, xb, we1t, be1r, we2t8, be2r)


def kernel(x, edge_index, edge_index_neg, W1, b1, W2, b2, We1, be1, We2, be2):
    xap, xbp = _gather(x, edge_index)
    xan, xbn = _gather(x, edge_index_neg)

    wcat_t = jnp.concatenate([W1, We1], axis=0).T
    w2t8 = jnp.pad(W2, ((0, 1), (0, 0))).T
    b2r = jnp.pad(b2, (0, 1)).reshape(1, 8)
    we2t8 = jnp.pad(We2, ((0, 7), (0, 0))).T
    b1r = b1.reshape(1, D)
    be1r = be1.reshape(1, D)
    be2r = be2.reshape(1, 1)
    attr8, pos8 = _decode_pos(xap, xbp, wcat_t, b1r, be1r, w2t8, b2r,
                              we2t8, be2r)
    neg8, = _decode_neg(xan, xbn, We1.T, be1r, we2t8, be2r)
    return attr8[:, :7], pos8[:, 0], neg8[:, 0]


# speedup vs baseline: 3.1772x; 1.2480x over previous
import functools

import jax
import jax.numpy as jnp
from jax import lax
from jax.experimental import pallas as pl
from jax.experimental.pallas import tpu as pltpu
from jax.experimental.pallas import tpu_sc as plsc

N = 10000
E = 320000
D = 128

_NC = 2
_NS = 16
_NW = _NC * _NS
_CHUNK = 80
_PER_W = E // _NW
_T = _PER_W // _CHUNK
_NBUF = 4


def _gather_body(x_hbm, ec, out, idx, ra, rb, wo, si, sga, sgb, swb):
    sid = lax.axis_index("s")
    wid = sid * _NC + lax.axis_index("c")
    w_base = wid * _PER_W

    def istart(t, b):
        base = w_base + t * _CHUNK
        pltpu.async_copy(ec.at[pl.ds(base, _CHUNK)], idx.at[b, 0], si.at[b])
        pltpu.async_copy(ec.at[pl.ds(E + base, _CHUNK)], idx.at[b, 1],
                         si.at[b])

    def iwait(t, b):
        base = w_base + t * _CHUNK
        pltpu.make_async_copy(ec.at[pl.ds(base, _CHUNK)], idx.at[b, 0],
                              si.at[b]).wait()
        pltpu.make_async_copy(ec.at[pl.ds(E + base, _CHUNK)], idx.at[b, 1],
                              si.at[b]).wait()

    def gstart(b):
        pltpu.async_copy(x_hbm.at[idx.at[b, 0]], ra.at[b], sga.at[b])
        pltpu.async_copy(x_hbm.at[idx.at[b, 1]], rb.at[b], sgb.at[b])

    def gwait(b):
        pltpu.make_async_copy(x_hbm.at[idx.at[b, 0]], ra.at[b],
                              sga.at[b]).wait()
        pltpu.make_async_copy(x_hbm.at[idx.at[b, 1]], rb.at[b],
                              sgb.at[b]).wait()

    def wbwait(t, b):
        sl = pl.ds(w_base + t * _CHUNK, _CHUNK)
        pltpu.make_async_copy(wo.at[b], out.at[sl], swb.at[b]).wait()

    def body(t, carry):
        b0 = lax.rem(t, _NBUF)
        b2 = lax.rem(t + 2, _NBUF)
        b3 = lax.rem(t + 3, _NBUF)

        @pl.when(t + 3 < _T)
        def _():
            istart(t + 3, b3)

        @pl.when(t + 2 < _T)
        def _():
            iwait(t + 2, b2)

            @pl.when(t >= 2)
            def _():
                wbwait(t - 2, b2)

            gstart(b2)

        gwait(b0)

        @plsc.parallel_loop(0, _CHUNK, unroll=4)
        def row_body(r):
            for k in range(D // 16):
                sl = pl.ds(k * 16, 16)
                wo[b0, r, sl] = ra[b0, r, sl] * rb[b0, r, sl]

        sl = pl.ds(w_base + t * _CHUNK, _CHUNK)
        pltpu.async_copy(wo.at[b0], out.at[sl], swb.at[b0])
        return carry

    for t in range(3):
        istart(t, t)
    for t in range(2):
        iwait(t, t)
        gstart(t)
    lax.fori_loop(0, _T, body, 0)
    for t in range(_T - _NBUF, _T):
        wbwait(t, t % _NBUF)


def _gather(x, ei):
    mesh = plsc.VectorSubcoreMesh(core_axis_name="c", subcore_axis_name="s")
    f = functools.partial(
        pl.kernel,
        mesh=mesh,
        out_type=jax.ShapeDtypeStruct((E, D), jnp.float32),
        scratch_types=[
            pltpu.VMEM((_NBUF, 2, _CHUNK), jnp.int32),
            pltpu.VMEM((_NBUF, _CHUNK, D), jnp.float32),
            pltpu.VMEM((_NBUF, _CHUNK, D), jnp.float32),
            pltpu.VMEM((_NBUF, _CHUNK, D), jnp.float32),
            pltpu.SemaphoreType.DMA((_NBUF,)),
            pltpu.SemaphoreType.DMA((_NBUF,)),
            pltpu.SemaphoreType.DMA((_NBUF,)),
            pltpu.SemaphoreType.DMA((_NBUF,)),
        ],
    )(_gather_body)
    return f(x, ei.reshape(2 * E))


_B = 4000
_G = E // _B

_EWISE = pl.BlockSpec((_B, D), lambda i: (i, 0))
_HEAD = pl.BlockSpec((_B, 8), lambda i: (i, 0))
_HEAD_SHAPE = jax.ShapeDtypeStruct((E, 8), jnp.float32)


def _decode_pos_body(em_ref, wcat_ref, b1_ref, be1_ref, w2t_ref,
                     b2_ref, we2t_ref, be2_ref, attr_ref, pos_ref):
    h = jnp.maximum(em_ref[...], 0.0)
    a = jnp.dot(h, wcat_ref[...])
    a1 = jnp.maximum(a[:, :D] + b1_ref[...], 0.0)
    attr_ref[...] = jax.nn.sigmoid(jnp.dot(a1, w2t_ref[...]) + b2_ref[...])
    ae = jnp.maximum(a[:, D:] + be1_ref[...], 0.0)
    pos_ref[...] = jax.nn.sigmoid(jnp.dot(ae, we2t_ref[...]) + be2_ref[...])


def _decode_neg_body(em_ref, we1t_ref, be1_ref, we2t_ref, be2_ref,
                     neg_ref):
    h = jnp.maximum(em_ref[...], 0.0)
    an = jnp.maximum(jnp.dot(h, we1t_ref[...]) + be1_ref[...], 0.0)
    neg_ref[...] = jax.nn.sigmoid(jnp.dot(an, we2t_ref[...]) + be2_ref[...])


def _decode_pos(em, wcat_t, b1r, be1r, w2t8, b2r, we2t8, be2r):
    return pl.pallas_call(
        _decode_pos_body,
        grid=(_G,),
        in_specs=[
            _EWISE,
            pl.BlockSpec((D, 2 * D), lambda i: (0, 0)),
            pl.BlockSpec((1, D), lambda i: (0, 0)),
            pl.BlockSpec((1, D), lambda i: (0, 0)),
            pl.BlockSpec((D, 8), lambda i: (0, 0)),
            pl.BlockSpec((1, 8), lambda i: (0, 0)),
            pl.BlockSpec((D, 8), lambda i: (0, 0)),
            pl.BlockSpec((1, 1), lambda i: (0, 0)),
        ],
        out_specs=[_HEAD, _HEAD],
        out_shape=[_HEAD_SHAPE, _HEAD_SHAPE],
        compiler_params=pltpu.CompilerParams(
            dimension_semantics=("arbitrary",),
        ),
    )(em, wcat_t, b1r, be1r, w2t8, b2r, we2t8, be2r)


def _decode_neg(em, we1t, be1r, we2t8, be2r):
    return pl.pallas_call(
        _decode_neg_body,
        grid=(_G,),
        in_specs=[
            _EWISE,
            pl.BlockSpec((D, D), lambda i: (0, 0)),
            pl.BlockSpec((1, D), lambda i: (0, 0)),
            pl.BlockSpec((D, 8), lambda i: (0, 0)),
            pl.BlockSpec((1, 1), lambda i: (0, 0)),
        ],
        out_specs=[_HEAD],
        out_shape=[_HEAD_SHAPE],
        compiler_params=pltpu.CompilerParams(
            dimension_semantics=("arbitrary",),
        ),
    )(em, we1t, be1r, we2t8, be2r)


def kernel(x, edge_index, edge_index_neg, W1, b1, W2, b2, We1, be1, We2, be2):
    em_pos = _gather(x, edge_index)
    em_neg = _gather(x, edge_index_neg)

    wcat_t = jnp.concatenate([W1, We1], axis=0).T
    w2t8 = jnp.pad(W2, ((0, 1), (0, 0))).T
    b2r = jnp.pad(b2, (0, 1)).reshape(1, 8)
    we2t8 = jnp.pad(We2, ((0, 7), (0, 0))).T
    b1r = b1.reshape(1, D)
    be1r = be1.reshape(1, D)
    be2r = be2.reshape(1, 1)
    attr8, pos8 = _decode_pos(em_pos, wcat_t, b1r, be1r, w2t8, b2r,
                              we2t8, be2r)
    neg8, = _decode_neg(em_neg, We1.T, be1r, we2t8, be2r)
    return attr8[:, :7], pos8[:, 0], neg8[:, 0]
